# Initial kernel scaffold; baseline (speedup 1.0000x reference)
import jax
import jax.numpy as jnp
from jax.experimental import pallas as pl


def kernel(x, tgt, lens, c, sos, labels):
    raise NotImplementedError("write your pallas kernel here")


# speedup vs baseline: 1.0299x; 1.0299x over previous
import jax
import jax.numpy as jnp
from jax.experimental import pallas as pl
from jax.experimental.pallas import tpu as pltpu

_B, _S, _J, _C, _TL = 16, 2048, 512, 10, 64
_TS = 512
_NT_IN = _S // _TS
_NT_X = (_S + 1 + _TS - 1) // _TS


def _onehot_replace(t, lb, sos_row, lab, base, start_off, nrows):
    rel0 = t * _TS - (lb + start_off)
    rows = jax.lax.broadcasted_iota(jnp.int32, (_TS, nrows), 0) + rel0
    cols = jax.lax.broadcasted_iota(jnp.int32, (_TS, nrows), 1)
    if nrows == 65:
        blk = jnp.concatenate([sos_row, lab], axis=0)
    else:
        blk = jnp.concatenate([sos_row, lab, sos_row], axis=0)
    oh = (rows == cols).astype(jnp.float32)
    repl = jax.lax.dot_general(
        oh, blk, (((1,), (0,)), ((), ())),
        precision=jax.lax.Precision.HIGHEST,
        preferred_element_type=jnp.float32)
    rel = rel0 + jax.lax.broadcasted_iota(jnp.int32, (_TS, 1), 0)
    mask = (rel >= 0) & (rel < nrows)
    return jnp.where(mask, repl, base)


def _x_body(lens_ref, c_ref, x_ref, sos_ref, lab_ref, o_ref, carry_ref):
    b = pl.program_id(0)
    t = pl.program_id(1)
    lb = lens_ref[b]
    xb = x_ref[0]

    @pl.when(t == 0)
    def _():
        carry_ref[...] = sos_ref[0]

    base = jnp.concatenate([carry_ref[...], xb[:-1]], axis=0)
    carry_ref[...] = xb[-1:]
    o_ref[0] = _onehot_replace(t, lb, sos_ref[0], lab_ref[0], base, 1, 65)


def _tgt_body(lens_ref, c_ref, tgt_ref, sos_ref, lab_ref, o_ref):
    b = pl.program_id(0)
    t = pl.program_id(1)
    lb = lens_ref[b]
    base = tgt_ref[0]
    o_ref[0] = _onehot_replace(t, lb, sos_ref[0], lab_ref[0], base, 0, 66)


def kernel(x, tgt, lens, c, sos, labels):
    sos3 = sos[:, None, :]
    x_grid = pltpu.PrefetchScalarGridSpec(
        num_scalar_prefetch=2,
        grid=(_B, _NT_X),
        in_specs=[
            pl.BlockSpec((1, _TS, _J),
                         lambda b, t, lens_ref, c_ref:
                         (b, jnp.minimum(t, _NT_IN - 1), 0)),
            pl.BlockSpec((1, 1, _J), lambda b, t, lens_ref, c_ref: (b, 0, 0)),
            pl.BlockSpec((1, _TL, _J),
                         lambda b, t, lens_ref, c_ref: (c_ref[b], 0, 0)),
        ],
        out_specs=pl.BlockSpec((1, _TS, _J),
                               lambda b, t, lens_ref, c_ref: (b, t, 0)),
        scratch_shapes=[pltpu.VMEM((1, _J), jnp.float32)],
    )
    out_x = pl.pallas_call(
        _x_body,
        grid_spec=x_grid,
        out_shape=jax.ShapeDtypeStruct((_B, _S + 1, _J), jnp.float32),
    )(lens, c, x, sos3, labels)

    t_grid = pltpu.PrefetchScalarGridSpec(
        num_scalar_prefetch=2,
        grid=(_B, _NT_IN),
        in_specs=[
            pl.BlockSpec((1, _TS, _J),
                         lambda b, t, lens_ref, c_ref: (b, t, 0)),
            pl.BlockSpec((1, 1, _J), lambda b, t, lens_ref, c_ref: (b, 0, 0)),
            pl.BlockSpec((1, _TL, _J),
                         lambda b, t, lens_ref, c_ref: (c_ref[b], 0, 0)),
        ],
        out_specs=pl.BlockSpec((1, _TS, _J),
                               lambda b, t, lens_ref, c_ref: (b, t, 0)),
    )
    out_tgt = pl.pallas_call(
        _tgt_body,
        grid_spec=t_grid,
        out_shape=jax.ShapeDtypeStruct((_B, _S, _J), jnp.float32),
    )(lens, c, tgt, sos3, labels)

    return (out_x, out_tgt, labels)


# speedup vs baseline: 1.1300x; 1.0973x over previous
import jax
import jax.numpy as jnp
from jax.experimental import pallas as pl
from jax.experimental.pallas import tpu as pltpu

_B, _S, _J, _C, _TL = 16, 2048, 512, 10, 64
_TS = 1024
_NT_IN = _S // _TS
_NT_X = (_S + 1 + _TS - 1) // _TS


def _onehot_replace(t, lb, sos_row, lab, base, start_off, nrows):
    rel0 = t * _TS - (lb + start_off)
    rows = jax.lax.broadcasted_iota(jnp.int32, (_TS, nrows), 0) + rel0
    cols = jax.lax.broadcasted_iota(jnp.int32, (_TS, nrows), 1)
    if nrows == 65:
        blk = jnp.concatenate([sos_row, lab], axis=0)
    else:
        blk = jnp.concatenate([sos_row, lab, sos_row], axis=0)
    oh = (rows == cols).astype(jnp.float32)
    repl = jax.lax.dot_general(
        oh, blk, (((1,), (0,)), ((), ())),
        precision=jax.lax.Precision.HIGHEST,
        preferred_element_type=jnp.float32)
    rel = rel0 + jax.lax.broadcasted_iota(jnp.int32, (_TS, 1), 0)
    mask = (rel >= 0) & (rel < nrows)
    return jnp.where(mask, repl, base)


def _x_body(lens_ref, c_ref, x_ref, sos_ref, lab_ref, o_ref, carry_ref):
    b = pl.program_id(0)
    t = pl.program_id(1)
    lb = lens_ref[b]
    xb = x_ref[0]

    @pl.when(t == 0)
    def _():
        carry_ref[...] = sos_ref[0]

    base = jnp.concatenate([carry_ref[...], xb[:-1]], axis=0)
    carry_ref[...] = xb[-1:]
    o_ref[0] = _onehot_replace(t, lb, sos_ref[0], lab_ref[0], base, 1, 65)


def _tgt_body(lens_ref, c_ref, tgt_ref, sos_ref, lab_ref, o_ref):
    b = pl.program_id(0)
    t = pl.program_id(1)
    lb = lens_ref[b]
    base = tgt_ref[0]
    o_ref[0] = _onehot_replace(t, lb, sos_ref[0], lab_ref[0], base, 0, 66)


def kernel(x, tgt, lens, c, sos, labels):
    sos3 = sos[:, None, :]
    x_grid = pltpu.PrefetchScalarGridSpec(
        num_scalar_prefetch=2,
        grid=(_B, _NT_X),
        in_specs=[
            pl.BlockSpec((1, _TS, _J),
                         lambda b, t, lens_ref, c_ref:
                         (b, jnp.minimum(t, _NT_IN - 1), 0)),
            pl.BlockSpec((1, 1, _J), lambda b, t, lens_ref, c_ref: (b, 0, 0)),
            pl.BlockSpec((1, _TL, _J),
                         lambda b, t, lens_ref, c_ref: (c_ref[b], 0, 0)),
        ],
        out_specs=pl.BlockSpec((1, _TS, _J),
                               lambda b, t, lens_ref, c_ref: (b, t, 0)),
        scratch_shapes=[pltpu.VMEM((1, _J), jnp.float32)],
    )
    out_x = pl.pallas_call(
        _x_body,
        grid_spec=x_grid,
        out_shape=jax.ShapeDtypeStruct((_B, _S + 1, _J), jnp.float32),
    )(lens, c, x, sos3, labels)

    t_grid = pltpu.PrefetchScalarGridSpec(
        num_scalar_prefetch=2,
        grid=(_B, _NT_IN),
        in_specs=[
            pl.BlockSpec((1, _TS, _J),
                         lambda b, t, lens_ref, c_ref: (b, t, 0)),
            pl.BlockSpec((1, 1, _J), lambda b, t, lens_ref, c_ref: (b, 0, 0)),
            pl.BlockSpec((1, _TL, _J),
                         lambda b, t, lens_ref, c_ref: (c_ref[b], 0, 0)),
        ],
        out_specs=pl.BlockSpec((1, _TS, _J),
                               lambda b, t, lens_ref, c_ref: (b, t, 0)),
    )
    out_tgt = pl.pallas_call(
        _tgt_body,
        grid_spec=t_grid,
        out_shape=jax.ShapeDtypeStruct((_B, _S, _J), jnp.float32),
    )(lens, c, tgt, sos3, labels)

    return (out_x, out_tgt, labels)
